# baseline (speedup 1.0000x reference)
import functools

import jax
import jax.numpy as jnp
from jax import lax
from jax.experimental import pallas as pl
from jax.experimental.pallas import tpu as pltpu
from jax.experimental.pallas import tpu_sc as plsc

NUM_CORES = 2
NUM_SUBCORES = 16
NW = NUM_CORES * NUM_SUBCORES
L = 16

IN_ROWS = 8192
OUT_ROWS = 32768
D = 1024
ROWS_PER_W = OUT_ROWS // NW
CHUNK = 32
N_CHUNKS = ROWS_PER_W // CHUNK
MAX_REP = 3


def _body(feeds_hbm, rt_hbm, cum_hbm, lim_hbm, out_hbm,
          rt_v, cum_v, idx_v, lim_v, rows0, rows1, sem0, sem1):
    wid = lax.axis_index("s") * NUM_CORES + lax.axis_index("c")
    base = wid * ROWS_PER_W

    scope1 = jax.named_scope("idx_build")
    scope1.__enter__()
    pltpu.sync_copy(rt_hbm, rt_v)
    pltpu.sync_copy(cum_hbm, cum_v)
    pltpu.sync_copy(lim_hbm, lim_v)
    limit = lim_v[...]

    def init(j, carry):
        idx_v[pl.ds(j * L, L)] = jnp.full((L,), IN_ROWS, jnp.int32)
        return carry

    lax.fori_loop(0, ROWS_PER_W // L, init, 0)

    lane = lax.iota(jnp.int32, L)

    def scan(j, carry):
        r = rt_v[pl.ds(j * L, L)]
        off = cum_v[pl.ds(j * L, L)] - r
        rowid = j * L + lane
        for k in range(MAX_REP):
            gpos = off + k
            pos = gpos - base
            mask = (r > k) & (pos >= 0) & (pos < ROWS_PER_W) & (gpos < limit)
            plsc.store_scatter(idx_v, [pos], rowid, mask=mask)
        return carry

    lax.fori_loop(0, IN_ROWS // L, scan, 0)
    scope1.__exit__(None, None, None)
    scope2 = jax.named_scope("row_gather")
    scope2.__enter__()

    rows = (rows0, rows1)
    sems = (sem0, sem1)

    def gather_desc(c, b):
        return pltpu.make_async_copy(
            feeds_hbm.at[idx_v.at[pl.ds(c * CHUNK, CHUNK)]], rows[b], sems[b])

    gather_desc(0, 0).start()
    gather_desc(1, 1).start()

    def pair(p, carry):
        for b in range(2):
            c = p * 2 + b
            gather_desc(c, b).wait()
            pltpu.sync_copy(rows[b], out_hbm.at[pl.ds(base + c * CHUNK, CHUNK)])
            cnext = jnp.minimum(c + 2, N_CHUNKS - 1)
            gather_desc(cnext, b).start()
        return carry

    lax.fori_loop(0, N_CHUNKS // 2, pair, 0)
    gather_desc(N_CHUNKS - 1, 0).wait()
    gather_desc(N_CHUNKS - 1, 1).wait()
    scope2.__exit__(None, None, None)


_sc_repeat = functools.partial(
    pl.kernel,
    out_type=jax.ShapeDtypeStruct((OUT_ROWS, D), jnp.float32),
    mesh=plsc.VectorSubcoreMesh(core_axis_name="c", subcore_axis_name="s"),
    compiler_params=pltpu.CompilerParams(needs_layout_passes=False),
    scratch_types=[
        pltpu.VMEM((IN_ROWS,), jnp.int32),
        pltpu.VMEM((IN_ROWS,), jnp.int32),
        pltpu.VMEM((ROWS_PER_W,), jnp.int32),
        pltpu.VMEM((L,), jnp.int32),
        pltpu.VMEM((CHUNK, D), jnp.float32),
        pltpu.VMEM((CHUNK, D), jnp.float32),
        pltpu.SemaphoreType.DMA,
        pltpu.SemaphoreType.DMA,
    ],
)(_body)


def kernel(feeds, feeds_repeat_times, output_feeds_size):
    rt = feeds_repeat_times.astype(jnp.int32)
    cum = jnp.cumsum(rt)
    limit = jnp.full((L,), jnp.minimum(output_feeds_size, OUT_ROWS), jnp.int32)
    feeds_ext = jnp.concatenate([feeds, jnp.zeros((8, D), feeds.dtype)], axis=0)
    return _sc_repeat(feeds_ext, rt, cum, limit)


# speedup vs baseline: 1.0459x; 1.0459x over previous
import functools

import jax
import jax.numpy as jnp
from jax import lax
from jax.experimental import pallas as pl
from jax.experimental.pallas import tpu as pltpu
from jax.experimental.pallas import tpu_sc as plsc

NUM_CORES = 2
NUM_SUBCORES = 16
NW = NUM_CORES * NUM_SUBCORES
L = 16

IN_ROWS = 8192
OUT_ROWS = 32768
D = 1024
ROWS_PER_W = OUT_ROWS // NW
CHUNK = 16
N_CHUNKS = ROWS_PER_W // CHUNK
NBUF = 4
LAG = 2
MAX_REP = 3


def _body(feeds_hbm, rt_hbm, cum_hbm, lim_hbm, out_hbm,
          rt_v, cum_v, idx_v, lim_v, rows, gsems, wsems):
    wid = lax.axis_index("s") * NUM_CORES + lax.axis_index("c")
    base = wid * ROWS_PER_W

    scope1 = jax.named_scope("idx_build")
    scope1.__enter__()
    pltpu.sync_copy(rt_hbm, rt_v)
    pltpu.sync_copy(cum_hbm, cum_v)
    pltpu.sync_copy(lim_hbm, lim_v)
    limit = lim_v[...]

    def init(j, carry):
        idx_v[pl.ds(j * L, L)] = jnp.full((L,), IN_ROWS, jnp.int32)
        return carry

    lax.fori_loop(0, ROWS_PER_W // L, init, 0)

    lane = lax.iota(jnp.int32, L)

    def scan(j, carry):
        r = rt_v[pl.ds(j * L, L)]
        off = cum_v[pl.ds(j * L, L)] - r
        rowid = j * L + lane
        for k in range(MAX_REP):
            gpos = off + k
            pos = gpos - base
            mask = (r > k) & (pos >= 0) & (pos < ROWS_PER_W) & (gpos < limit)
            plsc.store_scatter(idx_v, [pos], rowid, mask=mask)
        return carry

    lax.fori_loop(0, IN_ROWS // L, scan, 0)
    scope1.__exit__(None, None, None)
    scope2 = jax.named_scope("row_gather")
    scope2.__enter__()

    def gather_desc(c, b):
        return pltpu.make_async_copy(
            feeds_hbm.at[idx_v.at[pl.ds(c * CHUNK, CHUNK)]], rows[b], gsems[b])

    def wb_desc(c, b):
        return pltpu.make_async_copy(
            rows[b], out_hbm.at[pl.ds(base + c * CHUNK, CHUNK)], wsems[b])

    def step(c, j, drain_wb, issue_ahead):
        gather_desc(c, j).wait()
        wb_desc(c, j).start()
        if issue_ahead:
            b2 = (j + LAG) % NBUF
            if drain_wb:
                wb_desc(c + LAG - NBUF, b2).wait()
            gather_desc(c + LAG, b2).start()

    for c in range(LAG):
        gather_desc(c, c).start()

    for c in range(NBUF):
        step(c, c, drain_wb=(c + LAG >= NBUF), issue_ahead=True)

    def group(p, carry):
        c0 = (p + 1) * NBUF
        for j in range(NBUF):
            step(c0 + j, j, drain_wb=True, issue_ahead=True)
        return carry

    lax.fori_loop(0, N_CHUNKS // NBUF - 2, group, 0)

    for j in range(NBUF):
        c = N_CHUNKS - NBUF + j
        step(c, j, drain_wb=True, issue_ahead=(j < NBUF - LAG))
    for j in range(NBUF):
        wb_desc(N_CHUNKS - NBUF + j, j).wait()
    scope2.__exit__(None, None, None)


_sc_repeat = functools.partial(
    pl.kernel,
    out_type=jax.ShapeDtypeStruct((OUT_ROWS, D), jnp.float32),
    mesh=plsc.VectorSubcoreMesh(core_axis_name="c", subcore_axis_name="s"),
    compiler_params=pltpu.CompilerParams(needs_layout_passes=False),
    scratch_types=[
        pltpu.VMEM((IN_ROWS,), jnp.int32),
        pltpu.VMEM((IN_ROWS,), jnp.int32),
        pltpu.VMEM((ROWS_PER_W,), jnp.int32),
        pltpu.VMEM((L,), jnp.int32),
        tuple(pltpu.VMEM((CHUNK, D), jnp.float32) for _ in range(NBUF)),
        tuple(pltpu.SemaphoreType.DMA for _ in range(NBUF)),
        tuple(pltpu.SemaphoreType.DMA for _ in range(NBUF)),
    ],
)(_body)


def kernel(feeds, feeds_repeat_times, output_feeds_size):
    rt = feeds_repeat_times.astype(jnp.int32)
    cum = jnp.cumsum(rt)
    limit = jnp.full((L,), jnp.minimum(output_feeds_size, OUT_ROWS), jnp.int32)
    feeds_ext = jnp.concatenate([feeds, jnp.zeros((8, D), feeds.dtype)], axis=0)
    return _sc_repeat(feeds_ext, rt, cum, limit)


# speedup vs baseline: 4.9630x; 4.7453x over previous
import functools

import jax
import jax.numpy as jnp
from jax import lax
from jax.experimental import pallas as pl
from jax.experimental.pallas import tpu as pltpu
from jax.experimental.pallas import tpu_sc as plsc

NUM_CORES = 2
NUM_SUBCORES = 16
NW = NUM_CORES * NUM_SUBCORES
L = 16

IN_ROWS = 8192
OUT_ROWS = 32768
D = 1024
ROWS_PER_W = OUT_ROWS // NW
MAX_REP = 3

OUT_BLOCK = 1024
N_BLOCKS = OUT_ROWS // OUT_BLOCK


def _idx_body(rt_hbm, cum_hbm, lim_hbm, idx_hbm, rt_v, cum_v, idx_v, lim_v):
    wid = lax.axis_index("s") * NUM_CORES + lax.axis_index("c")
    base = wid * ROWS_PER_W

    pltpu.sync_copy(rt_hbm, rt_v)
    pltpu.sync_copy(cum_hbm, cum_v)
    pltpu.sync_copy(lim_hbm, lim_v)
    limit = lim_v[...]

    def init(j, carry):
        idx_v[pl.ds(j * L, L)] = jnp.full((L,), IN_ROWS, jnp.int32)
        return carry

    lax.fori_loop(0, ROWS_PER_W // L, init, 0)

    lane = lax.iota(jnp.int32, L)

    def scan(j, carry):
        r = rt_v[pl.ds(j * L, L)]
        off = cum_v[pl.ds(j * L, L)] - r
        rowid = j * L + lane
        for k in range(MAX_REP):
            gpos = off + k
            pos = gpos - base
            mask = (r > k) & (pos >= 0) & (pos < ROWS_PER_W) & (gpos < limit)
            plsc.store_scatter(idx_v, [pos], rowid, mask=mask)
        return carry

    lax.fori_loop(0, IN_ROWS // L, scan, 0)
    pltpu.sync_copy(idx_v, idx_hbm.at[pl.ds(base, ROWS_PER_W)])


_sc_idx = functools.partial(
    pl.kernel,
    out_type=jax.ShapeDtypeStruct((OUT_ROWS,), jnp.int32),
    mesh=plsc.VectorSubcoreMesh(core_axis_name="c", subcore_axis_name="s"),
    compiler_params=pltpu.CompilerParams(needs_layout_passes=False),
    scratch_types=[
        pltpu.VMEM((IN_ROWS,), jnp.int32),
        pltpu.VMEM((IN_ROWS,), jnp.int32),
        pltpu.VMEM((ROWS_PER_W,), jnp.int32),
        pltpu.VMEM((L,), jnp.int32),
    ],
)(_idx_body)


def _gather_tc(src_ref, feeds_hbm, out_ref, feeds_v, sem):
    step = pl.program_id(0)

    @pl.when(step == 0)
    def _():
        pltpu.make_async_copy(feeds_hbm, feeds_v.at[pl.ds(0, IN_ROWS)],
                              sem).start()
        feeds_v[IN_ROWS, :, :] = jnp.zeros((8, 128), jnp.float32)
        pltpu.make_async_copy(feeds_hbm, feeds_v.at[pl.ds(0, IN_ROWS)],
                              sem).wait()

    def copy_row(i, carry):
        s = src_ref[i]
        out_ref[pl.ds(i, 1)] = feeds_v[pl.ds(s, 1)]
        return carry

    lax.fori_loop(0, OUT_BLOCK, copy_row, 0, unroll=8)


_tc_gather = pl.pallas_call(
    _gather_tc,
    grid=(N_BLOCKS,),
    in_specs=[
        pl.BlockSpec((OUT_BLOCK,), lambda g: (g,),
                     memory_space=pltpu.SMEM),
        pl.BlockSpec(memory_space=pl.ANY),
    ],
    out_specs=pl.BlockSpec((OUT_BLOCK, 8, 128), lambda g: (g, 0, 0)),
    out_shape=jax.ShapeDtypeStruct((OUT_ROWS, 8, 128), jnp.float32),
    scratch_shapes=[
        pltpu.VMEM((IN_ROWS + 1, 8, 128), jnp.float32),
        pltpu.SemaphoreType.DMA,
    ],
)


def kernel(feeds, feeds_repeat_times, output_feeds_size):
    rt = feeds_repeat_times.astype(jnp.int32)
    cum = jnp.cumsum(rt)
    limit = jnp.full((L,), jnp.minimum(output_feeds_size, OUT_ROWS), jnp.int32)
    src = _sc_idx(rt, cum, limit)
    feeds_r = feeds.reshape(IN_ROWS, 8, 128)
    out = _tc_gather(src, feeds_r)
    return out.reshape(OUT_ROWS, D)


# speedup vs baseline: 7.4931x; 1.5098x over previous
import functools

import jax
import jax.numpy as jnp
from jax import lax
from jax.experimental import pallas as pl
from jax.experimental.pallas import tpu as pltpu
from jax.experimental.pallas import tpu_sc as plsc

NUM_CORES = 2
NUM_SUBCORES = 16
NW = NUM_CORES * NUM_SUBCORES
L = 16

IN_ROWS = 8192
OUT_ROWS = 32768
D = 1024
ROWS_PER_W = OUT_ROWS // NW
MAX_REP = 3

OUT_BLOCK = 1024
N_BLOCKS = OUT_ROWS // OUT_BLOCK


def _idx_body(rt_hbm, cum_hbm, lim_hbm, idx_hbm, rt_v, cum_v, idx_v, lim_v):
    wid = lax.axis_index("s") * NUM_CORES + lax.axis_index("c")
    base = wid * ROWS_PER_W

    pltpu.sync_copy(rt_hbm, rt_v)
    pltpu.sync_copy(cum_hbm, cum_v)
    pltpu.sync_copy(lim_hbm, lim_v)
    limit = lim_v[...]

    def init(j, carry):
        idx_v[pl.ds(j * L, L)] = jnp.full((L,), IN_ROWS, jnp.int32)
        return carry

    lax.fori_loop(0, ROWS_PER_W // L, init, 0)

    lane = lax.iota(jnp.int32, L)

    def scan(j, carry):
        r = rt_v[pl.ds(j * L, L)]
        off = cum_v[pl.ds(j * L, L)] - r
        rowid = j * L + lane
        for k in range(MAX_REP):
            gpos = off + k
            pos = gpos - base
            mask = (r > k) & (pos >= 0) & (pos < ROWS_PER_W) & (gpos < limit)
            plsc.store_scatter(idx_v, [pos], rowid, mask=mask)
        return carry

    lax.fori_loop(0, IN_ROWS // L, scan, 0)
    pltpu.sync_copy(idx_v, idx_hbm.at[pl.ds(base, ROWS_PER_W)])


_sc_idx = functools.partial(
    pl.kernel,
    out_type=jax.ShapeDtypeStruct((OUT_ROWS,), jnp.int32),
    mesh=plsc.VectorSubcoreMesh(core_axis_name="c", subcore_axis_name="s"),
    compiler_params=pltpu.CompilerParams(needs_layout_passes=False),
    scratch_types=[
        pltpu.VMEM((IN_ROWS,), jnp.int32),
        pltpu.VMEM((IN_ROWS,), jnp.int32),
        pltpu.VMEM((ROWS_PER_W,), jnp.int32),
        pltpu.VMEM((L,), jnp.int32),
    ],
)(_idx_body)


def _gather_tc(src_ref, feeds_hbm, out_ref, feeds_v, stage, sem):
    step = pl.program_id(0)

    @pl.when(step == 0)
    def _():
        pltpu.make_async_copy(feeds_hbm, feeds_v.at[pl.ds(0, IN_ROWS)],
                              sem).start()
        feeds_v[IN_ROWS, :, :] = jnp.zeros((8, 128), jnp.float32)
        pltpu.make_async_copy(feeds_hbm, feeds_v.at[pl.ds(0, IN_ROWS)],
                              sem).wait()

    def copy_row(i, carry):
        s = src_ref[i]
        stage[pl.ds(i, 1)] = feeds_v[pl.ds(s, 1)]
        return carry

    lax.fori_loop(0, OUT_BLOCK, copy_row, 0, unroll=8)
    out_ref[...] = stage[...].reshape(OUT_BLOCK, D)


_tc_gather = pl.pallas_call(
    _gather_tc,
    grid=(N_BLOCKS,),
    in_specs=[
        pl.BlockSpec((OUT_BLOCK,), lambda g: (g,),
                     memory_space=pltpu.SMEM),
        pl.BlockSpec(memory_space=pl.ANY),
    ],
    out_specs=pl.BlockSpec((OUT_BLOCK, D), lambda g: (g, 0)),
    out_shape=jax.ShapeDtypeStruct((OUT_ROWS, D), jnp.float32),
    scratch_shapes=[
        pltpu.VMEM((IN_ROWS + 1, 8, 128), jnp.float32),
        pltpu.VMEM((OUT_BLOCK, 8, 128), jnp.float32),
        pltpu.SemaphoreType.DMA,
    ],
)


def kernel(feeds, feeds_repeat_times, output_feeds_size):
    rt = feeds_repeat_times.astype(jnp.int32)
    cum = jnp.cumsum(rt)
    limit = jnp.full((L,), jnp.minimum(output_feeds_size, OUT_ROWS), jnp.int32)
    src = _sc_idx(rt, cum, limit)
    feeds_r = feeds.reshape(IN_ROWS, 8, 128)
    return _tc_gather(src, feeds_r)


# speedup vs baseline: 8.1365x; 1.0859x over previous
import functools

import jax
import jax.numpy as jnp
from jax import lax
from jax.experimental import pallas as pl
from jax.experimental.pallas import tpu as pltpu
from jax.experimental.pallas import tpu_sc as plsc

NUM_CORES = 2
NUM_SUBCORES = 16
NW = NUM_CORES * NUM_SUBCORES
L = 16

IN_ROWS = 8192
OUT_ROWS = 32768
D = 1024
ROWS_PER_W = OUT_ROWS // NW
MAX_REP = 3

OUT_BLOCK = 1024
N_BLOCKS = OUT_ROWS // OUT_BLOCK


def _idx_body(rt_hbm, cum_hbm, lim_hbm, idx_hbm, rt_v, cum_v, idx_v, lim_v):
    wid = lax.axis_index("s") * NUM_CORES + lax.axis_index("c")
    base = wid * ROWS_PER_W

    pltpu.sync_copy(rt_hbm, rt_v)
    pltpu.sync_copy(cum_hbm, cum_v)
    pltpu.sync_copy(lim_hbm, lim_v)
    limit = lim_v[...]

    def init(j, carry):
        idx_v[pl.ds(j * L, L)] = jnp.full((L,), IN_ROWS, jnp.int32)
        return carry

    lax.fori_loop(0, ROWS_PER_W // L, init, 0)

    lane = lax.iota(jnp.int32, L)

    def scan(j, carry):
        r = rt_v[pl.ds(j * L, L)]
        off = cum_v[pl.ds(j * L, L)] - r
        rowid = j * L + lane
        for k in range(MAX_REP):
            gpos = off + k
            pos = gpos - base
            mask = (r > k) & (pos >= 0) & (pos < ROWS_PER_W) & (gpos < limit)
            plsc.store_scatter(idx_v, [pos], rowid, mask=mask)
        return carry

    lax.fori_loop(0, IN_ROWS // L, scan, 0)
    pltpu.sync_copy(idx_v, idx_hbm.at[pl.ds(base, ROWS_PER_W)])


_sc_idx = functools.partial(
    pl.kernel,
    out_type=jax.ShapeDtypeStruct((OUT_ROWS,), jnp.int32),
    mesh=plsc.VectorSubcoreMesh(core_axis_name="c", subcore_axis_name="s"),
    compiler_params=pltpu.CompilerParams(needs_layout_passes=False),
    scratch_types=[
        pltpu.VMEM((IN_ROWS,), jnp.int32),
        pltpu.VMEM((IN_ROWS,), jnp.int32),
        pltpu.VMEM((ROWS_PER_W,), jnp.int32),
        pltpu.VMEM((L,), jnp.int32),
    ],
)(_idx_body)


def _gather_tc(src_ref, feeds_hbm, out_ref, feeds_v, stage, sem):
    step = pl.program_id(0)

    @pl.when(step == 0)
    def _():
        pltpu.make_async_copy(feeds_hbm, feeds_v.at[pl.ds(0, IN_ROWS)],
                              sem).start()
        feeds_v[IN_ROWS, :, :] = jnp.zeros((8, 128), jnp.float32)
        pltpu.make_async_copy(feeds_hbm, feeds_v.at[pl.ds(0, IN_ROWS)],
                              sem).wait()

    def copy_row(i, carry):
        s = src_ref[i]
        stage[pl.ds(i, 1)] = feeds_v[pl.ds(s, 1)]
        return carry

    lax.fori_loop(0, OUT_BLOCK, copy_row, 0, unroll=16)
    out_ref[...] = stage[...].reshape(OUT_BLOCK, D)


_tc_gather = pl.pallas_call(
    _gather_tc,
    grid=(N_BLOCKS,),
    in_specs=[
        pl.BlockSpec((OUT_BLOCK,), lambda g: (g,),
                     memory_space=pltpu.SMEM),
        pl.BlockSpec(memory_space=pl.ANY),
    ],
    out_specs=pl.BlockSpec((OUT_BLOCK, D), lambda g: (g, 0)),
    out_shape=jax.ShapeDtypeStruct((OUT_ROWS, D), jnp.float32),
    scratch_shapes=[
        pltpu.VMEM((IN_ROWS + 1, 8, 128), jnp.float32),
        pltpu.VMEM((OUT_BLOCK, 8, 128), jnp.float32),
        pltpu.SemaphoreType.DMA,
    ],
)


def kernel(feeds, feeds_repeat_times, output_feeds_size):
    rt = feeds_repeat_times.astype(jnp.int32)
    cum = jnp.cumsum(rt)
    limit = jnp.full((L,), jnp.minimum(output_feeds_size, OUT_ROWS), jnp.int32)
    src = _sc_idx(rt, cum, limit)
    feeds_r = feeds.reshape(IN_ROWS, 8, 128)
    return _tc_gather(src, feeds_r)


# speedup vs baseline: 9.1108x; 1.1198x over previous
import functools

import jax
import jax.numpy as jnp
from jax import lax
from jax.experimental import pallas as pl
from jax.experimental.pallas import tpu as pltpu
from jax.experimental.pallas import tpu_sc as plsc

NUM_CORES = 2
NUM_SUBCORES = 16
NW = NUM_CORES * NUM_SUBCORES
L = 16

IN_ROWS = 8192
OUT_ROWS = 32768
D = 1024
ROWS_PER_W = OUT_ROWS // NW
MAX_REP = 3

OUT_BLOCK = 1024
N_BLOCKS = OUT_ROWS // OUT_BLOCK


def _idx_body(rt_hbm, cum_hbm, lim_hbm, idx_hbm, rt_v, cum_v, idx_v, lim_v):
    wid = lax.axis_index("s") * NUM_CORES + lax.axis_index("c")
    base = wid * ROWS_PER_W

    pltpu.sync_copy(rt_hbm, rt_v)
    pltpu.sync_copy(cum_hbm, cum_v)
    pltpu.sync_copy(lim_hbm, lim_v)
    limit = lim_v[...]

    def init(j, carry):
        idx_v[pl.ds(j * L, L)] = jnp.full((L,), IN_ROWS, jnp.int32)
        return carry

    lax.fori_loop(0, ROWS_PER_W // L, init, 0)

    lane = lax.iota(jnp.int32, L)

    def scan(j, carry):
        r = rt_v[pl.ds(j * L, L)]
        off = cum_v[pl.ds(j * L, L)] - r
        rowid = j * L + lane
        for k in range(MAX_REP):
            gpos = off + k
            pos = gpos - base
            mask = (r > k) & (pos >= 0) & (pos < ROWS_PER_W) & (gpos < limit)
            plsc.store_scatter(idx_v, [pos], rowid, mask=mask)
        return carry

    lax.fori_loop(0, IN_ROWS // L, scan, 0)
    pltpu.sync_copy(idx_v, idx_hbm.at[pl.ds(base, ROWS_PER_W)])


_sc_idx = functools.partial(
    pl.kernel,
    out_type=jax.ShapeDtypeStruct((OUT_ROWS,), jnp.int32),
    mesh=plsc.VectorSubcoreMesh(core_axis_name="c", subcore_axis_name="s"),
    compiler_params=pltpu.CompilerParams(needs_layout_passes=False),
    scratch_types=[
        pltpu.VMEM((IN_ROWS,), jnp.int32),
        pltpu.VMEM((IN_ROWS,), jnp.int32),
        pltpu.VMEM((ROWS_PER_W,), jnp.int32),
        pltpu.VMEM((L,), jnp.int32),
    ],
)(_idx_body)


def _gather_tc(src_ref, feeds_hbm, out_ref, feeds_v, stage, sem):
    step = pl.program_id(0)

    @pl.when(step == 0)
    def _():
        pltpu.make_async_copy(feeds_hbm, feeds_v.at[pl.ds(0, IN_ROWS)],
                              sem).start()
        feeds_v[IN_ROWS, :, :] = jnp.zeros((8, 128), jnp.float32)
        pltpu.make_async_copy(feeds_hbm, feeds_v.at[pl.ds(0, IN_ROWS)],
                              sem).wait()

    all_pad = src_ref[0] == IN_ROWS

    @pl.when(all_pad)
    def _():
        out_ref[...] = jnp.zeros((OUT_BLOCK, D), jnp.float32)

    @pl.when(jnp.logical_not(all_pad))
    def _():
        def copy_row(i, carry):
            s = src_ref[i]
            stage[pl.ds(i, 1)] = feeds_v[pl.ds(s, 1)]
            return carry

        lax.fori_loop(0, OUT_BLOCK, copy_row, 0, unroll=16)
        out_ref[...] = stage[...].reshape(OUT_BLOCK, D)


_tc_gather = pl.pallas_call(
    _gather_tc,
    grid=(N_BLOCKS,),
    in_specs=[
        pl.BlockSpec((OUT_BLOCK,), lambda g: (g,),
                     memory_space=pltpu.SMEM),
        pl.BlockSpec(memory_space=pl.ANY),
    ],
    out_specs=pl.BlockSpec((OUT_BLOCK, D), lambda g: (g, 0)),
    out_shape=jax.ShapeDtypeStruct((OUT_ROWS, D), jnp.float32),
    scratch_shapes=[
        pltpu.VMEM((IN_ROWS + 1, 8, 128), jnp.float32),
        pltpu.VMEM((OUT_BLOCK, 8, 128), jnp.float32),
        pltpu.SemaphoreType.DMA,
    ],
)


def kernel(feeds, feeds_repeat_times, output_feeds_size):
    rt = feeds_repeat_times.astype(jnp.int32)
    cum = jnp.cumsum(rt)
    limit = jnp.full((L,), jnp.minimum(output_feeds_size, OUT_ROWS), jnp.int32)
    src = _sc_idx(rt, cum, limit)
    feeds_r = feeds.reshape(IN_ROWS, 8, 128)
    return _tc_gather(src, feeds_r)


# speedup vs baseline: 9.1686x; 1.0063x over previous
import functools

import jax
import jax.numpy as jnp
from jax import lax
from jax.experimental import pallas as pl
from jax.experimental.pallas import tpu as pltpu
from jax.experimental.pallas import tpu_sc as plsc

NUM_CORES = 2
NUM_SUBCORES = 16
NW = NUM_CORES * NUM_SUBCORES
L = 16

IN_ROWS = 8192
OUT_ROWS = 32768
D = 1024
ROWS_PER_W = OUT_ROWS // NW
MAX_REP = 3

OUT_BLOCK = 1024
N_BLOCKS = OUT_ROWS // OUT_BLOCK


def _idx_body(rt_hbm, cum_hbm, lim_hbm, idx_hbm, rt_v, cum_v, idx_v, lim_v):
    wid = lax.axis_index("s") * NUM_CORES + lax.axis_index("c")
    base = wid * ROWS_PER_W

    pltpu.sync_copy(rt_hbm, rt_v)
    pltpu.sync_copy(cum_hbm, cum_v)
    pltpu.sync_copy(lim_hbm, lim_v)
    limit = lim_v[...]

    def init(j, carry):
        idx_v[pl.ds(j * L, L)] = jnp.full((L,), IN_ROWS, jnp.int32)
        return carry

    lax.fori_loop(0, ROWS_PER_W // L, init, 0)

    lane = lax.iota(jnp.int32, L)

    def scan(j, carry):
        r = rt_v[pl.ds(j * L, L)]
        off = cum_v[pl.ds(j * L, L)] - r
        rowid = j * L + lane
        for k in range(MAX_REP):
            gpos = off + k
            pos = gpos - base
            mask = (r > k) & (pos >= 0) & (pos < ROWS_PER_W) & (gpos < limit)
            plsc.store_scatter(idx_v, [pos], rowid, mask=mask)
        return carry

    lax.fori_loop(0, IN_ROWS // L, scan, 0)
    pltpu.sync_copy(idx_v, idx_hbm.at[pl.ds(base, ROWS_PER_W)])


_sc_idx = functools.partial(
    pl.kernel,
    out_type=jax.ShapeDtypeStruct((OUT_ROWS,), jnp.int32),
    mesh=plsc.VectorSubcoreMesh(core_axis_name="c", subcore_axis_name="s"),
    compiler_params=pltpu.CompilerParams(needs_layout_passes=False),
    scratch_types=[
        pltpu.VMEM((IN_ROWS,), jnp.int32),
        pltpu.VMEM((IN_ROWS,), jnp.int32),
        pltpu.VMEM((ROWS_PER_W,), jnp.int32),
        pltpu.VMEM((L,), jnp.int32),
    ],
)(_idx_body)


def _gather_tc(src_ref, feeds_hbm, out_ref, feeds_v, stage, done, sem):
    step = pl.program_id(0)

    def feeds_dma():
        return pltpu.make_async_copy(feeds_hbm, feeds_v.at[pl.ds(0, IN_ROWS)],
                                     sem)

    @pl.when(step == 0)
    def _():
        feeds_dma().start()
        feeds_v[IN_ROWS, :, :] = jnp.zeros((8, 128), jnp.float32)
        done[0] = 0

    all_pad = src_ref[0] == IN_ROWS

    @pl.when(all_pad)
    def _():
        out_ref[...] = jnp.zeros((OUT_BLOCK, D), jnp.float32)

    @pl.when(jnp.logical_not(all_pad))
    def _():
        @pl.when(done[0] == 0)
        def _():
            feeds_dma().wait()
            done[0] = 1

        def copy_row(i, carry):
            s = src_ref[i]
            stage[pl.ds(i, 1)] = feeds_v[pl.ds(s, 1)]
            return carry

        lax.fori_loop(0, OUT_BLOCK, copy_row, 0, unroll=16)
        out_ref[...] = stage[...].reshape(OUT_BLOCK, D)

    @pl.when(jnp.logical_and(step == N_BLOCKS - 1, done[0] == 0))
    def _():
        feeds_dma().wait()
        done[0] = 1


_tc_gather = pl.pallas_call(
    _gather_tc,
    grid=(N_BLOCKS,),
    in_specs=[
        pl.BlockSpec((OUT_BLOCK,), lambda g: (N_BLOCKS - 1 - g,),
                     memory_space=pltpu.SMEM),
        pl.BlockSpec(memory_space=pl.ANY),
    ],
    out_specs=pl.BlockSpec((OUT_BLOCK, D), lambda g: (N_BLOCKS - 1 - g, 0)),
    out_shape=jax.ShapeDtypeStruct((OUT_ROWS, D), jnp.float32),
    scratch_shapes=[
        pltpu.VMEM((IN_ROWS + 1, 8, 128), jnp.float32),
        pltpu.VMEM((OUT_BLOCK, 8, 128), jnp.float32),
        pltpu.SMEM((1,), jnp.int32),
        pltpu.SemaphoreType.DMA,
    ],
)


def kernel(feeds, feeds_repeat_times, output_feeds_size):
    rt = feeds_repeat_times.astype(jnp.int32)
    cum = jnp.cumsum(rt)
    limit = jnp.full((L,), jnp.minimum(output_feeds_size, OUT_ROWS), jnp.int32)
    src = _sc_idx(rt, cum, limit)
    feeds_r = feeds.reshape(IN_ROWS, 8, 128)
    return _tc_gather(src, feeds_r)
